# baseline (speedup 1.0000x reference)
import functools

import jax
import jax.numpy as jnp
from jax import lax
from jax.experimental import pallas as pl
from jax.experimental.pallas import tpu as pltpu
from jax.experimental.pallas import tpu_sc as plsc

DIM = 64
SL = 128
NT = 200
NSEQ = 16384
NBT = NSEQ // SL
NC = 2
NS = 16
NW = NC * NS
BT_PER_W = NBT // NW
V = 10000
V_PER_TILE = V // NS


def _transpose_block(g, w, iota16):
    for bl0 in range(8):
        rowv = iota16 + bl0 * 16
        colv = jnp.zeros((16,), jnp.int32)
        one = jnp.ones((16,), jnp.int32)
        for d in range(DIM):
            vals = plsc.load_gather(g, [rowv, colv])
            w[d // 8, pl.ds((d % 8) * 128 + bl0 * 16, 16)] = vals
            colv = colv + one


@functools.lru_cache(maxsize=None)
def _make_kernel():
    mesh = plsc.VectorSubcoreMesh(core_axis_name="c", subcore_axis_name="s")
    n2 = NT // 2

    @functools.partial(
        pl.kernel,
        out_type=jax.ShapeDtypeStruct((NT, 8, NBT, 1024), jnp.float32),
        mesh=mesh,
        scratch_types=[
            pltpu.VMEM_SHARED((V, DIM), jnp.float32),
            pltpu.VMEM((NT, SL), jnp.int32),
            pltpu.VMEM((SL, DIM), jnp.float32),
            pltpu.VMEM((SL, DIM), jnp.float32),
            pltpu.VMEM((8, 1024), jnp.float32),
            pltpu.VMEM((8, 1024), jnp.float32),
            pltpu.SemaphoreType.DMA,
            pltpu.SemaphoreType.DMA,
            pltpu.SemaphoreType.DMA,
            pltpu.SemaphoreType.DMA,
        ],
        compiler_params=pltpu.CompilerParams(
            use_tc_tiling_on_sc=False, needs_layout_passes=False
        ),
    )
    def emb_kernel(idx_hbm, table_hbm, out_hbm, tab_sp, iv, g0, g1, w0, w1,
                   gs0, gs1, ss0, ss1):
        sid = lax.axis_index("s")
        wid = sid * NC + lax.axis_index("c")

        pltpu.sync_copy(
            table_hbm.at[pl.ds(sid * V_PER_TILE, V_PER_TILE)],
            tab_sp.at[pl.ds(sid * V_PER_TILE, V_PER_TILE)],
        )
        plsc.subcore_barrier()

        iota16 = lax.iota(jnp.int32, 16)

        def fire_gather(t, gv, sem):
            pltpu.async_copy(tab_sp.at[iv.at[t]], gv, sem)

        def wait_gather(t, gv, sem):
            pltpu.make_async_copy(tab_sp.at[iv.at[t]], gv, sem).wait()

        def start_store(wv, t, bt, sem):
            pltpu.async_copy(wv, out_hbm.at[t, :, bt], sem)

        def wait_store(wv, t, bt, sem):
            pltpu.make_async_copy(wv, out_hbm.at[t, :, bt], sem).wait()

        def kbody(k, kcarry):
            bt = wid * BT_PER_W + k
            pltpu.sync_copy(idx_hbm.at[bt], iv)
            fire_gather(0, g0, gs0)

            def body(i, carry):
                t0 = 2 * i
                t1 = t0 + 1
                fire_gather(t1, g1, gs1)
                wait_gather(t0, g0, gs0)

                @pl.when(i >= 1)
                def _():
                    wait_store(w0, t0 - 2, bt, ss0)

                _transpose_block(g0, w0, iota16)
                start_store(w0, t0, bt, ss0)

                @pl.when(t0 + 2 < NT)
                def _():
                    fire_gather(t0 + 2, g0, gs0)

                wait_gather(t1, g1, gs1)

                @pl.when(i >= 1)
                def _():
                    wait_store(w1, t1 - 2, bt, ss1)

                _transpose_block(g1, w1, iota16)
                start_store(w1, t1, bt, ss1)
                return carry

            lax.fori_loop(0, n2, body, 0)
            wait_store(w0, NT - 2, bt, ss0)
            wait_store(w1, NT - 1, bt, ss1)
            return kcarry

        lax.fori_loop(0, BT_PER_W, kbody, 0)

    return emb_kernel


@jax.jit
def kernel(input, table):
    idx3 = (
        input.astype(jnp.float32)
        .reshape(NBT, SL, NT)
        .transpose(0, 2, 1)
        .astype(jnp.int32)
    )
    out5 = _make_kernel()(idx3, table)
    return (
        out5.reshape(NT, 8, NBT, 8, SL)
        .transpose(2, 4, 0, 1, 3)
        .reshape(NSEQ, NT, DIM)
    )


# speedup vs baseline: 9.5403x; 9.5403x over previous
import functools

import jax
import jax.numpy as jnp
from jax import lax
from jax.experimental import pallas as pl
from jax.experimental.pallas import tpu as pltpu
from jax.experimental.pallas import tpu_sc as plsc

DIM = 64
SL = 128
NT = 200
NSEQ = 16384
NBT = NSEQ // SL
NC = 2
NS = 16
NW = NC * NS
BT_PER_W = NBT // NW
V = 10000
V_PER_TILE = V // NS


def _transpose_block(g, w1d, iota16, kvecs):
    def bl_body(bl0, c0):
        rowv = iota16 + bl0 * 16
        base16 = bl0 * 16

        def d_body(d0i, c1):
            d0 = d0i * 16
            wbase = iota16 + (d0 * 128 + base16)
            for j0 in (0, 8):
                vals = [
                    plsc.load_gather(g, [rowv, kvecs[j0 + j] + d0])
                    for j in range(8)
                ]
                for j in range(8):
                    plsc.store_scatter(
                        w1d, [wbase + kvecs[j0 + j] * 128], vals[j]
                    )
            return c1

        lax.fori_loop(0, DIM // 16, d_body, c0)
        return c0

    lax.fori_loop(0, 8, bl_body, 0)


@functools.lru_cache(maxsize=None)
def _make_kernel():
    mesh = plsc.VectorSubcoreMesh(core_axis_name="c", subcore_axis_name="s")
    n2 = NT // 2

    @functools.partial(
        pl.kernel,
        out_type=jax.ShapeDtypeStruct((NT, 8, NBT, 1024), jnp.float32),
        mesh=mesh,
        scratch_types=[
            pltpu.VMEM_SHARED((V, DIM), jnp.float32),
            pltpu.VMEM((NT, SL), jnp.int32),
            pltpu.VMEM((SL, DIM), jnp.float32),
            pltpu.VMEM((SL, DIM), jnp.float32),
            pltpu.VMEM((8 * 1024,), jnp.float32),
            pltpu.VMEM((8 * 1024,), jnp.float32),
            pltpu.SemaphoreType.DMA,
            pltpu.SemaphoreType.DMA,
            pltpu.SemaphoreType.DMA,
            pltpu.SemaphoreType.DMA,
        ],
        compiler_params=pltpu.CompilerParams(
            use_tc_tiling_on_sc=False, needs_layout_passes=False
        ),
    )
    def emb_kernel(idx_hbm, table_hbm, out_hbm, tab_sp, iv, g0, g1, w0, w1,
                   gs0, gs1, ss0, ss1):
        sid = lax.axis_index("s")
        wid = sid * NC + lax.axis_index("c")

        pltpu.sync_copy(
            table_hbm.at[pl.ds(sid * V_PER_TILE, V_PER_TILE)],
            tab_sp.at[pl.ds(sid * V_PER_TILE, V_PER_TILE)],
        )
        plsc.subcore_barrier()

        iota16 = lax.iota(jnp.int32, 16)
        kvecs = [(iota16 + j) & 15 for j in range(16)]

        def fire_gather(t, gv, sem):
            pltpu.async_copy(tab_sp.at[iv.at[t]], gv, sem)

        def wait_gather(t, gv, sem):
            pltpu.make_async_copy(tab_sp.at[iv.at[t]], gv, sem).wait()

        def start_store(wv, t, bt, sem):
            for dt in range(8):
                pltpu.async_copy(
                    wv.at[pl.ds(dt * 1024, 1024)],
                    out_hbm.at[t, dt, bt],
                    sem,
                )

        def wait_store(wv, t, bt, sem):
            for dt in range(8):
                pltpu.make_async_copy(
                    wv.at[pl.ds(dt * 1024, 1024)],
                    out_hbm.at[t, dt, bt],
                    sem,
                ).wait()

        def kbody(k, kcarry):
            bt = wid * BT_PER_W + k
            pltpu.sync_copy(idx_hbm.at[bt], iv)
            fire_gather(0, g0, gs0)

            def body(i, carry):
                t0 = 2 * i
                t1 = t0 + 1
                fire_gather(t1, g1, gs1)
                wait_gather(t0, g0, gs0)

                @pl.when(i >= 1)
                def _():
                    wait_store(w0, t0 - 2, bt, ss0)

                _transpose_block(g0, w0, iota16, kvecs)
                start_store(w0, t0, bt, ss0)

                @pl.when(t0 + 2 < NT)
                def _():
                    fire_gather(t0 + 2, g0, gs0)

                wait_gather(t1, g1, gs1)

                @pl.when(i >= 1)
                def _():
                    wait_store(w1, t1 - 2, bt, ss1)

                _transpose_block(g1, w1, iota16, kvecs)
                start_store(w1, t1, bt, ss1)
                return carry

            lax.fori_loop(0, n2, body, 0)
            wait_store(w0, NT - 2, bt, ss0)
            wait_store(w1, NT - 1, bt, ss1)
            return kcarry

        lax.fori_loop(0, BT_PER_W, kbody, 0)

    return emb_kernel


@jax.jit
def kernel(input, table):
    idx3 = (
        input.astype(jnp.float32)
        .reshape(NBT, SL, NT)
        .transpose(0, 2, 1)
        .astype(jnp.int32)
    )
    out5 = _make_kernel()(idx3, table)
    return (
        out5.reshape(NT, 8, NBT, 8, SL)
        .transpose(2, 4, 0, 1, 3)
        .reshape(NSEQ, NT, DIM)
    )


# speedup vs baseline: 9.5737x; 1.0035x over previous
import functools

import jax
import jax.numpy as jnp
from jax import lax
from jax.experimental import pallas as pl
from jax.experimental.pallas import tpu as pltpu
from jax.experimental.pallas import tpu_sc as plsc

DIM = 64
SL = 128
NT = 200
NSEQ = 16384
NBT = NSEQ // SL
NC = 2
NS = 16
NW = NC * NS
BT_PER_W = NBT // NW
V = 10000
V_PER_TILE = V // NS


def _transpose_block(g, w1d, iota16, kvecs, kvecs128):
    def bl_body(bl0, c0):
        rowv = iota16 + bl0 * 16
        base16 = bl0 * 16

        def d_body(d0i, c1):
            d0 = d0i * 16
            wbase = iota16 + (d0 * 128 + base16)
            for j0 in (0, 8):
                vals = [
                    plsc.load_gather(g, [rowv, kvecs[j0 + j] + d0])
                    for j in range(8)
                ]
                for j in range(8):
                    plsc.store_scatter(
                        w1d, [wbase + kvecs128[j0 + j]], vals[j]
                    )
            return c1

        lax.fori_loop(0, DIM // 16, d_body, c0)
        return c0

    lax.fori_loop(0, 8, bl_body, 0)


@functools.lru_cache(maxsize=None)
def _make_kernel():
    mesh = plsc.VectorSubcoreMesh(core_axis_name="c", subcore_axis_name="s")
    n2 = NT // 2

    @functools.partial(
        pl.kernel,
        out_type=jax.ShapeDtypeStruct((NT, 8, NBT, 1024), jnp.float32),
        mesh=mesh,
        scratch_types=[
            pltpu.VMEM_SHARED((V, DIM), jnp.float32),
            pltpu.VMEM((NT, SL), jnp.int32),
            pltpu.VMEM((SL, DIM), jnp.float32),
            pltpu.VMEM((SL, DIM), jnp.float32),
            pltpu.VMEM((8 * 1024,), jnp.float32),
            pltpu.VMEM((8 * 1024,), jnp.float32),
            pltpu.SemaphoreType.DMA,
            pltpu.SemaphoreType.DMA,
            pltpu.SemaphoreType.DMA,
            pltpu.SemaphoreType.DMA,
        ],
        compiler_params=pltpu.CompilerParams(
            use_tc_tiling_on_sc=False, needs_layout_passes=False
        ),
    )
    def emb_kernel(idx_hbm, table_hbm, out_hbm, tab_sp, iv, g0, g1, w0, w1,
                   gs0, gs1, ss0, ss1):
        sid = lax.axis_index("s")
        wid = sid * NC + lax.axis_index("c")

        pltpu.sync_copy(
            table_hbm.at[pl.ds(sid * V_PER_TILE, V_PER_TILE)],
            tab_sp.at[pl.ds(sid * V_PER_TILE, V_PER_TILE)],
        )
        plsc.subcore_barrier()

        iota16 = lax.iota(jnp.int32, 16)
        kvecs = [(iota16 + j) & 15 for j in range(16)]
        kvecs128 = [kv * 128 for kv in kvecs]

        def fire_gather(t, gv, sem):
            pltpu.async_copy(tab_sp.at[iv.at[t]], gv, sem)

        def wait_gather(t, gv, sem):
            pltpu.make_async_copy(tab_sp.at[iv.at[t]], gv, sem).wait()

        def start_store(wv, t, bt, sem):
            for dt in range(8):
                pltpu.async_copy(
                    wv.at[pl.ds(dt * 1024, 1024)],
                    out_hbm.at[t, dt, bt],
                    sem,
                )

        def wait_store(wv, t, bt, sem):
            for dt in range(8):
                pltpu.make_async_copy(
                    wv.at[pl.ds(dt * 1024, 1024)],
                    out_hbm.at[t, dt, bt],
                    sem,
                ).wait()

        def kbody(k, kcarry):
            bt = wid * BT_PER_W + k
            pltpu.sync_copy(idx_hbm.at[bt], iv)
            fire_gather(0, g0, gs0)

            def body(i, carry):
                t0 = 2 * i
                t1 = t0 + 1
                fire_gather(t1, g1, gs1)
                wait_gather(t0, g0, gs0)

                @pl.when(i >= 1)
                def _():
                    wait_store(w0, t0 - 2, bt, ss0)

                _transpose_block(g0, w0, iota16, kvecs, kvecs128)
                start_store(w0, t0, bt, ss0)

                @pl.when(t0 + 2 < NT)
                def _():
                    fire_gather(t0 + 2, g0, gs0)

                wait_gather(t1, g1, gs1)

                @pl.when(i >= 1)
                def _():
                    wait_store(w1, t1 - 2, bt, ss1)

                _transpose_block(g1, w1, iota16, kvecs, kvecs128)
                start_store(w1, t1, bt, ss1)
                return carry

            lax.fori_loop(0, n2, body, 0)
            wait_store(w0, NT - 2, bt, ss0)
            wait_store(w1, NT - 1, bt, ss1)
            return kcarry

        lax.fori_loop(0, BT_PER_W, kbody, 0)

    return emb_kernel


@jax.jit
def kernel(input, table):
    idx3 = (
        input.astype(jnp.float32)
        .reshape(NBT, SL, NT)
        .transpose(0, 2, 1)
        .astype(jnp.int32)
    )
    out5 = _make_kernel()(idx3, table)
    return (
        out5.reshape(NT, 8, NBT, 8, SL)
        .transpose(2, 4, 0, 1, 3)
        .reshape(NSEQ, NT, DIM)
    )


# speedup vs baseline: 9.6133x; 1.0041x over previous
import functools

import jax
import jax.numpy as jnp
from jax import lax
from jax.experimental import pallas as pl
from jax.experimental.pallas import tpu as pltpu
from jax.experimental.pallas import tpu_sc as plsc

DIM = 64
SL = 128
NT = 200
NSEQ = 16384
NBT = NSEQ // SL
NC = 2
NS = 16
NW = NC * NS
BT_PER_W = NBT // NW
V = 10000
V_PER_TILE = V // NS


def _transpose_block(g, w1d, iota16, kvecs, kvecs128):
    def bl_body(bl0, c0):
        rowv = iota16 + bl0 * 16
        base16 = bl0 * 16

        def d_body(d0i, c1):
            d0 = d0i * 16
            wbase = iota16 + (d0 * 128 + base16)
            for j0 in (0, 8):
                vals = [
                    plsc.load_gather(g, [rowv, kvecs[j0 + j] + d0])
                    for j in range(8)
                ]
                for j in range(8):
                    plsc.store_scatter(
                        w1d, [wbase + kvecs128[j0 + j]], vals[j]
                    )
            return c1

        lax.fori_loop(0, DIM // 16, d_body, c0)
        return c0

    lax.fori_loop(0, 8, bl_body, 0)


@functools.lru_cache(maxsize=None)
def _make_kernel():
    mesh = plsc.VectorSubcoreMesh(core_axis_name="c", subcore_axis_name="s")
    n2 = NT // 2

    @functools.partial(
        pl.kernel,
        out_type=jax.ShapeDtypeStruct((NT, 8, NBT, 1024), jnp.float32),
        mesh=mesh,
        scratch_types=[
            pltpu.VMEM_SHARED((V, DIM), jnp.float32),
            pltpu.VMEM((NT, SL), jnp.int32),
            pltpu.VMEM((SL, DIM), jnp.float32),
            pltpu.VMEM((SL, DIM), jnp.float32),
            pltpu.VMEM((8 * 1024,), jnp.float32),
            pltpu.VMEM((8 * 1024,), jnp.float32),
            pltpu.VMEM((8 * 1024,), jnp.float32),
            pltpu.VMEM((8 * 1024,), jnp.float32),
            pltpu.SemaphoreType.DMA,
            pltpu.SemaphoreType.DMA,
            pltpu.SemaphoreType.DMA,
            pltpu.SemaphoreType.DMA,
            pltpu.SemaphoreType.DMA,
            pltpu.SemaphoreType.DMA,
        ],
        compiler_params=pltpu.CompilerParams(
            use_tc_tiling_on_sc=False, needs_layout_passes=False
        ),
    )
    def emb_kernel(idx_hbm, table_hbm, out_hbm, tab_sp, iv, g0, g1,
                   w0, w1, w2, w3, gs0, gs1, ss0, ss1, ss2, ss3):
        sid = lax.axis_index("s")
        wid = sid * NC + lax.axis_index("c")

        pltpu.sync_copy(
            table_hbm.at[pl.ds(sid * V_PER_TILE, V_PER_TILE)],
            tab_sp.at[pl.ds(sid * V_PER_TILE, V_PER_TILE)],
        )
        plsc.subcore_barrier()

        iota16 = lax.iota(jnp.int32, 16)
        kvecs = [(iota16 + j) & 15 for j in range(16)]
        kvecs128 = [kv * 128 for kv in kvecs]

        def fire_gather(t, gv, sem):
            pltpu.async_copy(tab_sp.at[iv.at[t]], gv, sem)

        def wait_gather(t, gv, sem):
            pltpu.make_async_copy(tab_sp.at[iv.at[t]], gv, sem).wait()

        def start_store(wv, t, bt, sem):
            for dt in range(8):
                pltpu.async_copy(
                    wv.at[pl.ds(dt * 1024, 1024)],
                    out_hbm.at[t, dt, bt],
                    sem,
                )

        def wait_store(wv, t, bt, sem):
            for dt in range(8):
                pltpu.make_async_copy(
                    wv.at[pl.ds(dt * 1024, 1024)],
                    out_hbm.at[t, dt, bt],
                    sem,
                ).wait()

        gbufs = [(g0, gs0), (g1, gs1)]
        wbufs = [(w0, ss0), (w1, ss1), (w2, ss2), (w3, ss3)]
        n4 = NT // 4

        def kbody(k, kcarry):
            bt = wid * BT_PER_W + k
            pltpu.sync_copy(idx_hbm.at[bt], iv)
            fire_gather(0, g0, gs0)

            def body(i, carry):
                tb = 4 * i
                for step in range(4):
                    t = tb + step
                    gc, gcs = gbufs[step % 2]
                    gn, gns = gbufs[(step + 1) % 2]
                    wv, ws = wbufs[step]

                    @pl.when(t + 1 < NT)
                    def _():
                        fire_gather(t + 1, gn, gns)

                    wait_gather(t, gc, gcs)

                    @pl.when(i >= 1)
                    def _():
                        wait_store(wv, t - 4, bt, ws)

                    _transpose_block(gc, wv, iota16, kvecs, kvecs128)
                    start_store(wv, t, bt, ws)
                return carry

            lax.fori_loop(0, n4, body, 0)
            for step in range(4):
                wv, ws = wbufs[step]
                wait_store(wv, NT - 4 + step, bt, ws)
            return kcarry

        lax.fori_loop(0, BT_PER_W, kbody, 0)

    return emb_kernel


@jax.jit
def kernel(input, table):
    idx3 = (
        input.astype(jnp.float32)
        .reshape(NBT, SL, NT)
        .transpose(0, 2, 1)
        .astype(jnp.int32)
    )
    out5 = _make_kernel()(idx3, table)
    return (
        out5.reshape(NT, 8, NBT, 8, SL)
        .transpose(2, 4, 0, 1, 3)
        .reshape(NSEQ, NT, DIM)
    )
